# baseline (speedup 1.0000x reference)
import functools

import jax
import jax.numpy as jnp
from jax.experimental import pallas as pl

_NUM_EXPERTS = 64
_TOP_K = 2
_ALPHA = 0.01


def _gate_body(nj, scale, hs_ref, wt_ref, i1_ref, i2_ref, w1_ref, w2_ref,
               psum_ref, fsum_ref, aux_ref):
    b = pl.program_id(0)
    j = pl.program_id(1)
    hs = hs_ref[0]
    logits = jax.lax.dot_general(
        hs, wt_ref[...], (((1,), (0,)), ((), ())),
        preferred_element_type=jnp.float32)
    m = jnp.max(logits, axis=-1, keepdims=True)
    e = jnp.exp(logits - m)
    s = e / jnp.sum(e, axis=-1, keepdims=True)

    iota = jax.lax.broadcasted_iota(jnp.int32, s.shape, 1)
    m1 = jnp.max(s, axis=-1, keepdims=True)
    i1 = jnp.min(jnp.where(s == m1, iota, _NUM_EXPERTS), axis=-1)
    masked = jnp.where(iota == i1[:, None], -1.0, s)
    m2 = jnp.max(masked, axis=-1, keepdims=True)
    i2 = jnp.min(jnp.where(masked == m2, iota, _NUM_EXPERTS), axis=-1)

    i1_ref[0] = i1[None]
    i2_ref[0] = i2[None]
    w1_ref[0] = m1[:, 0][None]
    w2_ref[0] = m2[:, 0][None]

    cnt = (jnp.sum((iota == i1[:, None]).astype(jnp.float32), axis=0)
           + jnp.sum((iota == i2[:, None]).astype(jnp.float32), axis=0))

    @pl.when(j == 0)
    def _():
        psum_ref[...] = jnp.zeros_like(psum_ref)
        fsum_ref[...] = jnp.zeros_like(fsum_ref)

    psum_ref[...] += jnp.sum(s, axis=0)[None, None]
    fsum_ref[...] += cnt[None, None]

    @pl.when(j == nj - 1)
    def _():
        term = jnp.sum(psum_ref[0, 0] * fsum_ref[0, 0]) * scale

        @pl.when(b == 0)
        def _():
            aux_ref[...] = jnp.full((1, 1), 0.0, jnp.float32) + term

        @pl.when(b > 0)
        def _():
            aux_ref[...] += term


def kernel(hidden_states, weight):
    B, S, H = hidden_states.shape
    E = _NUM_EXPERTS
    TS = 512
    nj = S // TS
    scale = _ALPHA * E / (B * _TOP_K * float(S) ** 3)
    wt = weight.T

    grid = (B, nj)
    out_shapes = (
        jax.ShapeDtypeStruct((B, 1, S), jnp.int32),
        jax.ShapeDtypeStruct((B, 1, S), jnp.int32),
        jax.ShapeDtypeStruct((B, 1, S), jnp.float32),
        jax.ShapeDtypeStruct((B, 1, S), jnp.float32),
        jax.ShapeDtypeStruct((B, 1, E), jnp.float32),
        jax.ShapeDtypeStruct((B, 1, E), jnp.float32),
        jax.ShapeDtypeStruct((1, 1), jnp.float32),
    )
    plane = pl.BlockSpec((1, 1, TS), lambda b, j: (b, 0, j))
    be = pl.BlockSpec((1, 1, E), lambda b, j: (b, 0, 0))
    i1, i2, w1, w2, _psum, _fsum, aux = pl.pallas_call(
        functools.partial(_gate_body, nj, scale),
        grid=grid,
        in_specs=[
            pl.BlockSpec((1, TS, H), lambda b, j: (b, j, 0)),
            pl.BlockSpec((H, E), lambda b, j: (0, 0)),
        ],
        out_specs=(plane, plane, plane, plane, be, be,
                   pl.BlockSpec((1, 1), lambda b, j: (0, 0))),
        out_shape=out_shapes,
    )(hidden_states, wt)

    topk_idx = jnp.stack([i1[:, 0, :], i2[:, 0, :]], axis=-1)
    topk_weight = jnp.stack([w1[:, 0, :], w2[:, 0, :]], axis=-1)
    aux_loss = aux[0, 0]
    return topk_idx, topk_weight, aux_loss


# speedup vs baseline: 2.6955x; 2.6955x over previous
import functools

import jax
import jax.numpy as jnp
from jax.experimental import pallas as pl

_NUM_EXPERTS = 64
_TOP_K = 2
_ALPHA = 0.01


def _gate_body(nj, scale, hs_ref, w_ref, i1_ref, i2_ref, w1_ref, w2_ref,
               psum_ref, fsum_ref, aux_ref):
    b = pl.program_id(0)
    j = pl.program_id(1)
    ts = hs_ref.shape[1]
    hs = hs_ref[0]
    logits = jax.lax.dot_general(
        hs, w_ref[...], (((1,), (0,)), ((), ())),
        preferred_element_type=jnp.float32)
    lt = logits.T

    m = jnp.max(lt, axis=0, keepdims=True)
    e = jnp.exp(lt - m)
    sig = jnp.sum(e, axis=0, keepdims=True)
    s = e / sig

    iota = jax.lax.broadcasted_iota(jnp.int32, s.shape, 0)
    m1 = jnp.max(s, axis=0, keepdims=True)
    i1 = jnp.min(jnp.where(s == m1, iota, _NUM_EXPERTS), axis=0)
    masked = jnp.where(iota == i1[None, :], -1.0, s)
    m2 = jnp.max(masked, axis=0, keepdims=True)
    i2 = jnp.min(jnp.where(masked == m2, iota, _NUM_EXPERTS), axis=0)

    i1_ref[0] = i1[None]
    i2_ref[0] = i2[None]
    w1_ref[0] = m1
    w2_ref[0] = m2

    ones = jnp.ones((ts, 1), jnp.float32)
    cnt_eq = ((iota == i1[None, :]).astype(jnp.float32)
              + (iota == i2[None, :]).astype(jnp.float32))
    psum_part = jax.lax.dot_general(
        s, ones, (((1,), (0,)), ((), ())),
        preferred_element_type=jnp.float32)
    fsum_part = jax.lax.dot_general(
        cnt_eq, ones, (((1,), (0,)), ((), ())),
        preferred_element_type=jnp.float32)

    @pl.when(j == 0)
    def _():
        psum_ref[...] = jnp.zeros_like(psum_ref)
        fsum_ref[...] = jnp.zeros_like(fsum_ref)

    psum_ref[...] += psum_part[None]
    fsum_ref[...] += fsum_part[None]

    @pl.when(j == nj - 1)
    def _():
        term = jnp.sum(psum_ref[0, :, 0] * fsum_ref[0, :, 0]) * scale

        @pl.when(b == 0)
        def _():
            aux_ref[...] = jnp.full((1, 1), 0.0, jnp.float32) + term

        @pl.when(b > 0)
        def _():
            aux_ref[...] += term


def kernel(hidden_states, weight):
    B, S, H = hidden_states.shape
    E = _NUM_EXPERTS
    TS = 512
    nj = S // TS
    scale = _ALPHA * E / (B * _TOP_K * float(S) ** 3)
    wt = weight.T

    grid = (B, nj)
    out_shapes = (
        jax.ShapeDtypeStruct((B, 1, S), jnp.int32),
        jax.ShapeDtypeStruct((B, 1, S), jnp.int32),
        jax.ShapeDtypeStruct((B, 1, S), jnp.float32),
        jax.ShapeDtypeStruct((B, 1, S), jnp.float32),
        jax.ShapeDtypeStruct((B, E, 1), jnp.float32),
        jax.ShapeDtypeStruct((B, E, 1), jnp.float32),
        jax.ShapeDtypeStruct((1, 1), jnp.float32),
    )
    plane = pl.BlockSpec((1, 1, TS), lambda b, j: (b, 0, j))
    be = pl.BlockSpec((1, E, 1), lambda b, j: (b, 0, 0))
    i1, i2, w1, w2, _psum, _fsum, aux = pl.pallas_call(
        functools.partial(_gate_body, nj, scale),
        grid=grid,
        in_specs=[
            pl.BlockSpec((1, TS, H), lambda b, j: (b, j, 0)),
            pl.BlockSpec((H, E), lambda b, j: (0, 0)),
        ],
        out_specs=(plane, plane, plane, plane, be, be,
                   pl.BlockSpec((1, 1), lambda b, j: (0, 0))),
        out_shape=out_shapes,
    )(hidden_states, wt)

    topk_idx = jnp.stack([i1[:, 0, :], i2[:, 0, :]], axis=-1)
    topk_weight = jnp.stack([w1[:, 0, :], w2[:, 0, :]], axis=-1)
    aux_loss = aux[0, 0]
    return topk_idx, topk_weight, aux_loss


# speedup vs baseline: 3.8264x; 1.4195x over previous
import functools

import jax
import jax.numpy as jnp
from jax.experimental import pallas as pl

_NUM_EXPERTS = 64
_TOP_K = 2
_ALPHA = 0.01


def _gate_body(nj, scale, hs_ref, w_ref, i1_ref, i2_ref, w1_ref, w2_ref,
               psum_ref, fsum_ref, aux_ref):
    b = pl.program_id(0)
    j = pl.program_id(1)
    ts = hs_ref.shape[1]
    hs = hs_ref[0]
    lt = jax.lax.dot_general(
        w_ref[...], hs, (((1,), (1,)), ((), ())),
        preferred_element_type=jnp.float32)

    m = jnp.max(lt, axis=0, keepdims=True)
    e = jnp.exp(lt - m)
    sig = jnp.sum(e, axis=0, keepdims=True)
    recip = 1.0 / sig
    s = e * recip

    iota = jax.lax.broadcasted_iota(jnp.int32, s.shape, 0)
    m1 = jnp.max(s, axis=0, keepdims=True)
    i1 = jnp.min(jnp.where(s == m1, iota, _NUM_EXPERTS), axis=0)
    masked = jnp.where(iota == i1[None, :], -1.0, s)
    m2 = jnp.max(masked, axis=0, keepdims=True)
    i2 = jnp.min(jnp.where(masked == m2, iota, _NUM_EXPERTS), axis=0)

    i1_ref[0] = i1[None]
    i2_ref[0] = i2[None]
    w1_ref[0] = m1
    w2_ref[0] = m2

    ones = jnp.ones((ts, 1), jnp.float32)
    cnt_eq = ((iota == i1[None, :]).astype(jnp.float32)
              + (iota == i2[None, :]).astype(jnp.float32))
    psum_part = jax.lax.dot_general(
        s, ones, (((1,), (0,)), ((), ())),
        preferred_element_type=jnp.float32)
    fsum_part = jax.lax.dot_general(
        cnt_eq, ones, (((1,), (0,)), ((), ())),
        preferred_element_type=jnp.float32)

    @pl.when(j == 0)
    def _():
        psum_ref[...] = jnp.zeros_like(psum_ref)
        fsum_ref[...] = jnp.zeros_like(fsum_ref)

    psum_ref[...] += psum_part[None]
    fsum_ref[...] += fsum_part[None]

    @pl.when(j == nj - 1)
    def _():
        term = jnp.sum(psum_ref[0, :, 0] * fsum_ref[0, :, 0]) * scale

        @pl.when(b == 0)
        def _():
            aux_ref[...] = jnp.full((1, 1), 0.0, jnp.float32) + term

        @pl.when(b > 0)
        def _():
            aux_ref[...] += term


def kernel(hidden_states, weight):
    B, S, H = hidden_states.shape
    E = _NUM_EXPERTS
    TS = 1024
    nj = S // TS
    scale = _ALPHA * E / (B * _TOP_K * float(S) ** 3)

    grid = (B, nj)
    out_shapes = (
        jax.ShapeDtypeStruct((B, 1, S), jnp.int32),
        jax.ShapeDtypeStruct((B, 1, S), jnp.int32),
        jax.ShapeDtypeStruct((B, 1, S), jnp.float32),
        jax.ShapeDtypeStruct((B, 1, S), jnp.float32),
        jax.ShapeDtypeStruct((B, E, 1), jnp.float32),
        jax.ShapeDtypeStruct((B, E, 1), jnp.float32),
        jax.ShapeDtypeStruct((1, 1), jnp.float32),
    )
    plane = pl.BlockSpec((1, 1, TS), lambda b, j: (b, 0, j))
    be = pl.BlockSpec((1, E, 1), lambda b, j: (b, 0, 0))
    i1, i2, w1, w2, _psum, _fsum, aux = pl.pallas_call(
        functools.partial(_gate_body, nj, scale),
        grid=grid,
        in_specs=[
            pl.BlockSpec((1, TS, H), lambda b, j: (b, j, 0)),
            pl.BlockSpec((E, H), lambda b, j: (0, 0)),
        ],
        out_specs=(plane, plane, plane, plane, be, be,
                   pl.BlockSpec((1, 1), lambda b, j: (0, 0))),
        out_shape=out_shapes,
    )(hidden_states, weight)

    topk_idx = jnp.stack([i1[:, 0, :], i2[:, 0, :]], axis=-1)
    topk_weight = jnp.stack([w1[:, 0, :], w2[:, 0, :]], axis=-1)
    aux_loss = aux[0, 0]
    return topk_idx, topk_weight, aux_loss
